# Initial kernel scaffold; baseline (speedup 1.0000x reference)
import jax
import jax.numpy as jnp
from jax.experimental import pallas as pl


def kernel(idx, targets, token_embedding_table):
    raise NotImplementedError("write your pallas kernel here")


# speedup vs baseline: 1.3661x; 1.3661x over previous
import functools

import jax
import jax.numpy as jnp
from jax import lax
from jax.experimental import pallas as pl
from jax.experimental.pallas import tpu as pltpu
from jax.experimental.pallas import tpu_sc as plsc

V = 1000
BT = 1024 * 50
NC, NS, L = 2, 16, 16
NW = NC * NS
B_PER_W = BT // NW
CHUNK = 32
NCHUNK = B_PER_W // CHUNK


def _lse_body(tab_ref, lse_ref):
    x = tab_ref[...]
    m = jnp.max(x, axis=1, keepdims=True)
    s = jnp.sum(jnp.exp(x - m), axis=1, keepdims=True)
    lse_ref[...] = m + jnp.log(s)


def _row_lse(table):
    return pl.pallas_call(
        _lse_body,
        out_shape=jax.ShapeDtypeStruct((V, 1), jnp.float32),
    )(table)


_mesh = plsc.VectorSubcoreMesh(core_axis_name="c", subcore_axis_name="s")


@functools.partial(
    pl.kernel,
    mesh=_mesh,
    compiler_params=pltpu.CompilerParams(needs_layout_passes=False,
                                         use_tc_tiling_on_sc=False),
    out_type=[
        jax.ShapeDtypeStruct((BT, V), jnp.float32),
        jax.ShapeDtypeStruct((NW, L), jnp.float32),
    ],
    scratch_types=[
        pltpu.VMEM((NCHUNK, CHUNK), jnp.int32),
        pltpu.VMEM((NCHUNK, CHUNK), jnp.int32),
        pltpu.VMEM((CHUNK, V), jnp.float32),
        pltpu.VMEM((8, 128), jnp.float32),
        pltpu.VMEM((L,), jnp.float32),
        pltpu.SemaphoreType.DMA,
    ],
)
def _sc_gather(table_hbm, idx_hbm, tgt_hbm, lse_hbm, out_hbm, part_hbm,
               idx_v, tgt_v, rows_v, lse_v, acc_v, sem):
    wid = lax.axis_index("s") * NC + lax.axis_index("c")
    base = wid * B_PER_W
    pltpu.sync_copy(idx_hbm.at[wid], idx_v)
    pltpu.sync_copy(tgt_hbm.at[wid], tgt_v)
    pltpu.sync_copy(lse_hbm, lse_v)

    def body(c, acc):
        idx_row = idx_v.at[c]
        pltpu.async_copy(table_hbm.at[idx_row], rows_v, sem).wait()
        pltpu.sync_copy(rows_v, out_hbm.at[pl.ds(base + c * CHUNK, CHUNK)])
        for g in range(CHUNK // L):
            rid = lax.broadcasted_iota(jnp.int32, (L,), 0) + g * L
            vids = idx_row[pl.ds(g * L, L)]
            tg = tgt_v.at[c][pl.ds(g * L, L)]
            lz = plsc.load_gather(lse_v, [vids >> 7, vids & 127])
            picked = plsc.load_gather(rows_v, [rid, tg])
            acc = acc + (lz - picked)
        return acc

    acc = lax.fori_loop(0, NCHUNK, body, jnp.zeros((L,), jnp.float32))
    acc_v[...] = acc
    pltpu.sync_copy(acc_v, part_hbm.at[wid])


def kernel(idx, targets, token_embedding_table):
    Bb, Tt = idx.shape
    lse = jnp.pad(_row_lse(token_embedding_table).reshape(V),
                  (0, 8 * 128 - V)).reshape(8, 128)
    idx3 = idx.reshape(NW, NCHUNK, CHUNK).astype(jnp.int32)
    tgt3 = targets.reshape(NW, NCHUNK, CHUNK).astype(jnp.int32)
    out, parts = _sc_gather(token_embedding_table, idx3, tgt3, lse)
    logits = out.reshape(Bb, Tt, V)
    loss = jnp.sum(parts) / jnp.float32(BT)
    return (logits, loss)


# speedup vs baseline: 1.3963x; 1.0222x over previous
import functools

import jax
import jax.numpy as jnp
from jax import lax
from jax.experimental import pallas as pl
from jax.experimental.pallas import tpu as pltpu
from jax.experimental.pallas import tpu_sc as plsc

V = 1000
B = 1024
T = 50
BT = B * T
NC, NS, L = 2, 16, 16
NW = NC * NS
B_PER_W = B // NW
NGRP = (T + L - 1) // L


def _lse_body(tab_ref, lse_ref):
    x = tab_ref[...]
    m = jnp.max(x, axis=1, keepdims=True)
    s = jnp.sum(jnp.exp(x - m), axis=1, keepdims=True)
    lse_ref[...] = m + jnp.log(s)


def _row_lse(table):
    return pl.pallas_call(
        _lse_body,
        out_shape=jax.ShapeDtypeStruct((V, 1), jnp.float32),
    )(table)


_mesh = plsc.VectorSubcoreMesh(core_axis_name="c", subcore_axis_name="s")


@functools.partial(
    pl.kernel,
    mesh=_mesh,
    compiler_params=pltpu.CompilerParams(needs_layout_passes=False,
                                         use_tc_tiling_on_sc=False),
    out_type=[
        jax.ShapeDtypeStruct((B, T, V), jnp.float32),
        jax.ShapeDtypeStruct((NW, L), jnp.float32),
    ],
    scratch_types=[
        pltpu.VMEM((B_PER_W, T), jnp.int32),
        pltpu.VMEM((B_PER_W, T), jnp.int32),
        pltpu.VMEM((T, V), jnp.float32),
        pltpu.VMEM((8, 128), jnp.float32),
        pltpu.VMEM((L,), jnp.float32),
        pltpu.SemaphoreType.DMA,
    ],
)
def _sc_gather(table_hbm, idx_hbm, tgt_hbm, lse_hbm, out_hbm, part_hbm,
               idx_v, tgt_v, rows_v, lse_v, acc_v, sem):
    wid = lax.axis_index("s") * NC + lax.axis_index("c")
    base = wid * B_PER_W
    pltpu.sync_copy(idx_hbm.at[wid], idx_v)
    pltpu.sync_copy(tgt_hbm.at[wid], tgt_v)
    pltpu.sync_copy(lse_hbm, lse_v)

    def body(c, acc):
        idx_row = idx_v.at[c]
        pltpu.async_copy(table_hbm.at[idx_row], rows_v, sem).wait()
        pltpu.sync_copy(rows_v, out_hbm.at[base + c])
        for g in range(NGRP):
            rid = lax.broadcasted_iota(jnp.int32, (L,), 0) + g * L
            live = rid < T
            rid = jnp.minimum(rid, T - 1)
            vids = plsc.load_gather(idx_v.at[c], [rid])
            tg = plsc.load_gather(tgt_v.at[c], [rid])
            lz = plsc.load_gather(lse_v, [vids >> 7, vids & 127])
            picked = plsc.load_gather(rows_v, [rid, tg])
            acc = acc + jnp.where(live, lz - picked, 0.0)
        return acc

    acc = lax.fori_loop(0, B_PER_W, body, jnp.zeros((L,), jnp.float32))
    acc_v[...] = acc
    pltpu.sync_copy(acc_v, part_hbm.at[wid])


def kernel(idx, targets, token_embedding_table):
    lse = jnp.pad(_row_lse(token_embedding_table).reshape(V),
                  (0, 8 * 128 - V)).reshape(8, 128)
    idx3 = idx.reshape(NW, B_PER_W, T).astype(jnp.int32)
    tgt3 = targets.reshape(NW, B_PER_W, T).astype(jnp.int32)
    logits, parts = _sc_gather(token_embedding_table, idx3, tgt3, lse)
    loss = jnp.sum(parts) / jnp.float32(BT)
    return (logits, loss)


# speedup vs baseline: 1.6577x; 1.1872x over previous
import functools

import jax
import jax.numpy as jnp
from jax import lax
from jax.experimental import pallas as pl
from jax.experimental.pallas import tpu as pltpu
from jax.experimental.pallas import tpu_sc as plsc

V = 1000
B = 1024
T = 50
BT = B * T
NC, NS, L = 2, 16, 16
NW = NC * NS
CR = V // 8
P_PER_W = BT // NW
NPICK = ((P_PER_W + 127) // 128) * 128


def _lse_body(tab_ref, lse_ref):
    x = tab_ref[...]
    m = jnp.max(x, axis=1, keepdims=True)
    s = jnp.sum(jnp.exp(x - m), axis=1, keepdims=True)
    lse_ref[...] = m + jnp.log(s)


def _row_lse(table):
    return pl.pallas_call(
        _lse_body,
        out_shape=jax.ShapeDtypeStruct((V, 1), jnp.float32),
    )(table)


_mesh = plsc.VectorSubcoreMesh(core_axis_name="c", subcore_axis_name="s")


@functools.partial(
    pl.kernel,
    mesh=_mesh,
    compiler_params=pltpu.CompilerParams(needs_layout_passes=False,
                                         use_tc_tiling_on_sc=False),
    out_type=[
        jax.ShapeDtypeStruct((T, CR, 8, 8, 128), jnp.float32),
        jax.ShapeDtypeStruct((NW, L), jnp.float32),
    ],
    scratch_types=[
        pltpu.VMEM((T, B), jnp.int32),
        pltpu.VMEM((8, V), jnp.float32),
        pltpu.VMEM((2, 8, 8, 128), jnp.float32),
        pltpu.VMEM((8, 128), jnp.float32),
        pltpu.VMEM((P_PER_W,), jnp.int32),
        pltpu.VMEM((P_PER_W,), jnp.int32),
        pltpu.VMEM((NPICK,), jnp.int32),
        pltpu.VMEM((NPICK,), jnp.float32),
        pltpu.VMEM((L,), jnp.float32),
        pltpu.SemaphoreType.DMA,
        pltpu.SemaphoreType.DMA,
        pltpu.SemaphoreType.DMA,
    ],
)
def _sc_writer(tflat_hbm, idxt_hbm, idxf_hbm, tgtf_hbm, lse_hbm,
               out_hbm, part_hbm,
               idxt_v, tblk_v, stage_v, lse_v, idxw_v, tgtw_v, pick_v,
               picked_v, acc_v, sem_a, sem_b, sem_m):
    wid = lax.axis_index("s") * NC + lax.axis_index("c")

    pltpu.sync_copy(idxf_hbm.at[wid], idxw_v)
    pltpu.sync_copy(tgtf_hbm.at[wid], tgtw_v)
    pltpu.sync_copy(lse_hbm, lse_v)

    def mkpick(j, _):
        i16 = idxw_v[pl.ds(j * L, L)]
        t16 = tgtw_v[pl.ds(j * L, L)]
        pick_v[pl.ds(j * L, L)] = t16 * V + i16
        return 0

    lax.fori_loop(0, P_PER_W // L, mkpick, 0)
    zero16 = jnp.zeros((L,), jnp.int32)
    for g in range((NPICK - P_PER_W) // L):
        pick_v[pl.ds(P_PER_W + g * L, L)] = zero16

    def pickgather(j, _):
        pltpu.async_copy(tflat_hbm.at[pick_v.at[pl.ds(j * 128, 128)]],
                         picked_v.at[pl.ds(j * 128, 128)], sem_m).wait()
        return 0

    lax.fori_loop(0, NPICK // 128, pickgather, 0)

    def lossacc(j, acc):
        vids = idxw_v[pl.ds(j * L, L)]
        lz = plsc.load_gather(lse_v, [vids >> 7, vids & 127])
        pk = picked_v[pl.ds(j * L, L)]
        return acc + (lz - pk)

    acc = lax.fori_loop(0, P_PER_W // L, lossacc, jnp.zeros((L,), jnp.float32))
    acc_v[...] = acc
    pltpu.sync_copy(acc_v, part_hbm.at[wid])

    pltpu.sync_copy(idxt_hbm, idxt_v)
    cr_lo = (wid * CR) // NW
    cr_hi = ((wid + 1) * CR) // NW

    def fill(t, buf):
        sb = stage_v.at[buf]
        irow = idxt_v.at[t]
        for bc in range(8):
            vids = [irow[pl.ds(bc * 128 + g * L, L)] for g in range(8)]
            for ci in range(8):
                row = tblk_v.at[ci]
                for g in range(8):
                    sb[bc, ci, pl.ds(g * L, L)] = plsc.load_gather(
                        row, [vids[g]])

    def cr_body(cr, _):
        for ci in range(8):
            pltpu.sync_copy(tflat_hbm.at[pl.ds((cr * 8 + ci) * V, V)],
                            tblk_v.at[ci])
        kbase = (cr - cr_lo) * T

        def t_body(tp, _):
            for par, sem in ((0, sem_a), (1, sem_b)):
                t = 2 * tp + par
                k = kbase + t

                @pl.when(k >= 2)
                def _():
                    pltpu.make_async_copy(
                        stage_v.at[par], out_hbm.at[t, cr], sem).wait()

                fill(t, par)
                pltpu.async_copy(stage_v.at[par], out_hbm.at[t, cr], sem)
            return 0

        lax.fori_loop(0, T // 2, t_body, 0)
        return 0

    lax.fori_loop(cr_lo, cr_hi, cr_body, 0)
    last = cr_hi - 1
    pltpu.make_async_copy(stage_v.at[0], out_hbm.at[T - 2, last], sem_a).wait()
    pltpu.make_async_copy(stage_v.at[1], out_hbm.at[T - 1, last], sem_b).wait()


def kernel(idx, targets, token_embedding_table):
    idx = idx.astype(jnp.int32)
    targets = targets.astype(jnp.int32)
    lse = jnp.pad(_row_lse(token_embedding_table).reshape(V),
                  (0, 8 * 128 - V)).reshape(8, 128)
    ttab = token_embedding_table.T
    tflat = ttab.reshape(V * V)
    idxt = idx.T
    idxf = idx.reshape(NW, P_PER_W)
    tgtf = targets.reshape(NW, P_PER_W)
    out5, parts = _sc_writer(tflat, idxt, idxf, tgtf, lse)
    logits = jnp.transpose(out5, (2, 4, 0, 1, 3)).reshape(B, T, V)
    loss = jnp.sum(parts) / jnp.float32(BT)
    return (logits, loss)


# speedup vs baseline: 2.5524x; 1.5397x over previous
import functools

import jax
import jax.numpy as jnp
from jax import lax
from jax.experimental import pallas as pl
from jax.experimental.pallas import tpu as pltpu
from jax.experimental.pallas import tpu_sc as plsc

V = 1000
B = 1024
T = 50
BT = B * T
NC, NS, L = 2, 16, 16
NW = NC * NS
CR = V // 8
P_PER_W = BT // NW
NPICK = ((P_PER_W + 127) // 128) * 128


def _lse_body(tab_ref, lse_ref):
    x = tab_ref[...]
    m = jnp.max(x, axis=1, keepdims=True)
    s = jnp.sum(jnp.exp(x - m), axis=1, keepdims=True)
    lse_ref[...] = m + jnp.log(s)


def _row_lse(table):
    return pl.pallas_call(
        _lse_body,
        out_shape=jax.ShapeDtypeStruct((V, 1), jnp.float32),
    )(table)


_mesh = plsc.VectorSubcoreMesh(core_axis_name="c", subcore_axis_name="s")


@functools.partial(
    pl.kernel,
    mesh=_mesh,
    compiler_params=pltpu.CompilerParams(needs_layout_passes=False,
                                         use_tc_tiling_on_sc=False),
    out_type=[
        jax.ShapeDtypeStruct((T, CR, 8, 8, 128), jnp.float32),
        jax.ShapeDtypeStruct((NW, L), jnp.float32),
    ],
    scratch_types=[
        pltpu.VMEM((T, B), jnp.int32),
        pltpu.VMEM((8, V), jnp.float32),
        pltpu.VMEM((2, 8, 8, 128), jnp.float32),
        pltpu.VMEM((8, 128), jnp.float32),
        pltpu.VMEM((P_PER_W,), jnp.int32),
        pltpu.VMEM((P_PER_W,), jnp.int32),
        pltpu.VMEM((NPICK,), jnp.int32),
        pltpu.VMEM((NPICK,), jnp.float32),
        pltpu.VMEM((L,), jnp.float32),
        pltpu.SemaphoreType.DMA,
        pltpu.SemaphoreType.DMA,
        pltpu.SemaphoreType.DMA,
    ],
)
def _sc_writer(tflat_hbm, idxt_hbm, idxf_hbm, tgtf_hbm, lse_hbm,
               out_hbm, part_hbm,
               idxt_v, tblk_v, stage_v, lse_v, idxw_v, tgtw_v, pick_v,
               picked_v, acc_v, sem_a, sem_b, sem_m):
    wid = lax.axis_index("s") * NC + lax.axis_index("c")

    pltpu.sync_copy(idxf_hbm.at[wid], idxw_v)
    pltpu.sync_copy(tgtf_hbm.at[wid], tgtw_v)
    pltpu.sync_copy(lse_hbm, lse_v)

    def mkpick(j, _):
        i16 = idxw_v[pl.ds(j * L, L)]
        t16 = tgtw_v[pl.ds(j * L, L)]
        pick_v[pl.ds(j * L, L)] = t16 * V + i16
        return 0

    lax.fori_loop(0, P_PER_W // L, mkpick, 0)
    zero16 = jnp.zeros((L,), jnp.int32)
    for g in range((NPICK - P_PER_W) // L):
        pick_v[pl.ds(P_PER_W + g * L, L)] = zero16

    def pickgather(j, _):
        pltpu.async_copy(tflat_hbm.at[pick_v.at[pl.ds(j * 128, 128)]],
                         picked_v.at[pl.ds(j * 128, 128)], sem_m).wait()
        return 0

    lax.fori_loop(0, NPICK // 128, pickgather, 0)

    def lossacc(j, acc):
        vids = idxw_v[pl.ds(j * L, L)]
        lz = plsc.load_gather(lse_v, [vids >> 7, vids & 127])
        pk = picked_v[pl.ds(j * L, L)]
        return acc + (lz - pk)

    acc = lax.fori_loop(0, P_PER_W // L, lossacc, jnp.zeros((L,), jnp.float32))
    acc_v[...] = acc
    pltpu.sync_copy(acc_v, part_hbm.at[wid])

    pltpu.sync_copy(idxt_hbm, idxt_v)
    cr_lo = (wid * CR) // NW
    cr_hi = ((wid + 1) * CR) // NW

    def fill(t, buf):
        sb = stage_v.at[buf]
        irow = idxt_v.at[t]
        for bc in range(8):
            vids = [irow[pl.ds(bc * 128 + g * L, L)] for g in range(8)]
            for g in range(8):
                vals = [plsc.load_gather(tblk_v.at[ci], [vids[g]])
                        for ci in range(8)]
                for ci in range(8):
                    sb[bc, ci, pl.ds(g * L, L)] = vals[ci]

    def cr_body(cr, _):
        for ci in range(8):
            pltpu.sync_copy(tflat_hbm.at[pl.ds((cr * 8 + ci) * V, V)],
                            tblk_v.at[ci])
        kbase = (cr - cr_lo) * T

        def t_body(tp, _):
            for par, sem in ((0, sem_a), (1, sem_b)):
                t = 2 * tp + par
                k = kbase + t

                @pl.when(k >= 2)
                def _():
                    pltpu.make_async_copy(
                        stage_v.at[par], out_hbm.at[t, cr], sem).wait()

                fill(t, par)
                pltpu.async_copy(stage_v.at[par], out_hbm.at[t, cr], sem)
            return 0

        lax.fori_loop(0, T // 2, t_body, 0)
        return 0

    lax.fori_loop(cr_lo, cr_hi, cr_body, 0)
    last = cr_hi - 1
    pltpu.make_async_copy(stage_v.at[0], out_hbm.at[T - 2, last], sem_a).wait()
    pltpu.make_async_copy(stage_v.at[1], out_hbm.at[T - 1, last], sem_b).wait()


def kernel(idx, targets, token_embedding_table):
    idx = idx.astype(jnp.int32)
    targets = targets.astype(jnp.int32)
    lse = jnp.pad(_row_lse(token_embedding_table).reshape(V),
                  (0, 8 * 128 - V)).reshape(8, 128)
    ttab = token_embedding_table.T
    tflat = ttab.reshape(V * V)
    idxt = idx.T
    idxf = idx.reshape(NW, P_PER_W)
    tgtf = targets.reshape(NW, P_PER_W)
    out5, parts = _sc_writer(tflat, idxt, idxf, tgtf, lse)
    logits = jnp.transpose(out5, (2, 4, 0, 1, 3)).reshape(B, T, V)
    loss = jnp.sum(parts) / jnp.float32(BT)
    return (logits, loss)


# speedup vs baseline: 5.6164x; 2.2004x over previous
import functools

import jax
import jax.numpy as jnp
from jax import lax
from jax.experimental import pallas as pl
from jax.experimental.pallas import tpu as pltpu
from jax.experimental.pallas import tpu_sc as plsc

V = 1000
B = 1024
T = 50
BT = B * T
NC, NS, L = 2, 16, 16
NW = NC * NS
CR = V // 8
P_PER_W = BT // NW
NPICK = ((P_PER_W + 127) // 128) * 128


def _lse_body(tab_ref, lse_ref):
    x = tab_ref[...]
    m = jnp.max(x, axis=1, keepdims=True)
    s = jnp.sum(jnp.exp(x - m), axis=1, keepdims=True)
    lse_ref[...] = m + jnp.log(s)


def _row_lse(table):
    return pl.pallas_call(
        _lse_body,
        out_shape=jax.ShapeDtypeStruct((V, 1), jnp.float32),
    )(table)


_mesh = plsc.VectorSubcoreMesh(core_axis_name="c", subcore_axis_name="s")


@functools.partial(
    pl.kernel,
    mesh=_mesh,
    compiler_params=pltpu.CompilerParams(needs_layout_passes=False,
                                         use_tc_tiling_on_sc=False),
    out_type=[
        jax.ShapeDtypeStruct((T, CR, 8, 8, 128), jnp.float32),
        jax.ShapeDtypeStruct((NW, L), jnp.float32),
    ],
    scratch_types=[
        pltpu.VMEM((T, B), jnp.int32),
        pltpu.VMEM((8, V), jnp.float32),
        pltpu.VMEM((2, 8, 8, 128), jnp.float32),
        pltpu.VMEM((8, 128), jnp.float32),
        pltpu.VMEM((P_PER_W,), jnp.int32),
        pltpu.VMEM((P_PER_W,), jnp.int32),
        pltpu.VMEM((NPICK,), jnp.int32),
        pltpu.VMEM((NPICK,), jnp.float32),
        pltpu.VMEM((L,), jnp.float32),
        pltpu.SemaphoreType.DMA,
        pltpu.SemaphoreType.DMA,
        pltpu.SemaphoreType.DMA,
    ],
)
def _sc_writer(tflat_hbm, idxt_hbm, idxf_hbm, tgtf_hbm, lse_hbm,
               out_hbm, part_hbm,
               idxt_v, tblk_v, stage_v, lse_v, idxw_v, tgtw_v, pick_v,
               picked_v, acc_v, sem_a, sem_b, sem_m):
    wid = lax.axis_index("s") * NC + lax.axis_index("c")

    pltpu.sync_copy(idxf_hbm.at[wid], idxw_v)
    pltpu.sync_copy(tgtf_hbm.at[wid], tgtw_v)
    pltpu.sync_copy(lse_hbm, lse_v)

    def mkpick(j, _):
        i16 = idxw_v[pl.ds(j * L, L)]
        t16 = tgtw_v[pl.ds(j * L, L)]
        pick_v[pl.ds(j * L, L)] = t16 * V + i16
        return 0

    lax.fori_loop(0, P_PER_W // L, mkpick, 0)
    zero16 = jnp.zeros((L,), jnp.int32)
    for g in range((NPICK - P_PER_W) // L):
        pick_v[pl.ds(P_PER_W + g * L, L)] = zero16

    def pickgather(j, _):
        pltpu.async_copy(tflat_hbm.at[pick_v.at[pl.ds(j * 128, 128)]],
                         picked_v.at[pl.ds(j * 128, 128)], sem_m).wait()
        return 0

    lax.fori_loop(0, NPICK // 128, pickgather, 0)

    def lossacc(j, acc):
        vids = idxw_v[pl.ds(j * L, L)]
        lz = plsc.load_gather(lse_v, [vids >> 7, vids & 127])
        pk = picked_v[pl.ds(j * L, L)]
        return acc + (lz - pk)

    acc = lax.fori_loop(0, P_PER_W // L, lossacc, jnp.zeros((L,), jnp.float32))
    acc_v[...] = acc
    pltpu.sync_copy(acc_v, part_hbm.at[wid])

    pltpu.sync_copy(idxt_hbm, idxt_v)
    cr_lo = (wid * CR) // NW
    cr_hi = ((wid + 1) * CR) // NW

    def fill(t, buf):
        sb = stage_v.at[buf]
        irow = idxt_v.at[t]
        for bc in range(8):
            vids = [irow[pl.ds(bc * 128 + g * L, L)] for g in range(8)]
            prev = None
            for g in range(8):
                vals = []
                for ci in range(8):
                    vals.append(plsc.load_gather(tblk_v.at[ci], [vids[g]]))
                    if prev is not None:
                        sb[bc, ci, pl.ds((g - 1) * L, L)] = prev[ci]
                prev = vals
            for ci in range(8):
                sb[bc, ci, pl.ds(7 * L, L)] = prev[ci]

    def cr_body(cr, _):
        for ci in range(8):
            pltpu.sync_copy(tflat_hbm.at[pl.ds((cr * 8 + ci) * V, V)],
                            tblk_v.at[ci])
        kbase = (cr - cr_lo) * T

        def t_body(tp, _):
            for par, sem in ((0, sem_a), (1, sem_b)):
                t = 2 * tp + par
                k = kbase + t

                @pl.when(k >= 2)
                def _():
                    pltpu.make_async_copy(
                        stage_v.at[par], out_hbm.at[t, cr], sem).wait()

                fill(t, par)
                pltpu.async_copy(stage_v.at[par], out_hbm.at[t, cr], sem)
            return 0

        lax.fori_loop(0, T // 2, t_body, 0)
        return 0

    lax.fori_loop(cr_lo, cr_hi, cr_body, 0)
    last = cr_hi - 1
    pltpu.make_async_copy(stage_v.at[0], out_hbm.at[T - 2, last], sem_a).wait()
    pltpu.make_async_copy(stage_v.at[1], out_hbm.at[T - 1, last], sem_b).wait()


def kernel(idx, targets, token_embedding_table):
    idx = idx.astype(jnp.int32)
    targets = targets.astype(jnp.int32)
    lse = jnp.pad(_row_lse(token_embedding_table).reshape(V),
                  (0, 8 * 128 - V)).reshape(8, 128)
    ttab = token_embedding_table.T
    tflat = ttab.reshape(V * V)
    idxt = idx.T
    idxf = idx.reshape(NW, P_PER_W)
    tgtf = targets.reshape(NW, P_PER_W)
    out5, parts = _sc_writer(tflat, idxt, idxf, tgtf, lse)
    logits = jnp.transpose(out5, (2, 4, 0, 1, 3)).reshape(B, T, V)
    loss = jnp.sum(parts) / jnp.float32(BT)
    return (logits, loss)


# speedup vs baseline: 5.9185x; 1.0538x over previous
import functools

import jax
import jax.numpy as jnp
from jax import lax
from jax.experimental import pallas as pl
from jax.experimental.pallas import tpu as pltpu
from jax.experimental.pallas import tpu_sc as plsc

V = 1000
B = 1024
T = 50
BT = B * T
NC, NS, L = 2, 16, 16
NW = NC * NS
CR = V // 8
P_PER_W = BT // NW
NPICK = ((P_PER_W + 127) // 128) * 128


def _lse_body(tab_ref, lse_ref):
    x = tab_ref[...]
    m = jnp.max(x, axis=1, keepdims=True)
    s = jnp.sum(jnp.exp(x - m), axis=1, keepdims=True)
    lse_ref[...] = m + jnp.log(s)


def _row_lse(table):
    return pl.pallas_call(
        _lse_body,
        out_shape=jax.ShapeDtypeStruct((V, 1), jnp.float32),
    )(table)


_mesh = plsc.VectorSubcoreMesh(core_axis_name="c", subcore_axis_name="s")


@functools.partial(
    pl.kernel,
    mesh=_mesh,
    compiler_params=pltpu.CompilerParams(needs_layout_passes=False,
                                         use_tc_tiling_on_sc=False),
    out_type=[
        jax.ShapeDtypeStruct((T, CR, 8, 8, 128), jnp.float32),
        jax.ShapeDtypeStruct((NW, L), jnp.float32),
    ],
    scratch_types=[
        pltpu.VMEM((T, B), jnp.int32),
        pltpu.VMEM((8, V), jnp.float32),
        pltpu.VMEM((2, 8, 8, 128), jnp.float32),
        pltpu.VMEM((8, 128), jnp.float32),
        pltpu.VMEM((P_PER_W,), jnp.int32),
        pltpu.VMEM((P_PER_W,), jnp.int32),
        pltpu.VMEM((NPICK,), jnp.int32),
        pltpu.VMEM((NPICK,), jnp.float32),
        pltpu.VMEM((L,), jnp.float32),
        pltpu.SemaphoreType.DMA,
        pltpu.SemaphoreType.DMA,
        pltpu.SemaphoreType.DMA,
        pltpu.SemaphoreType.DMA,
    ],
)
def _sc_writer(tflat_hbm, idxt_hbm, idxf_hbm, tgtf_hbm, lse_hbm,
               out_hbm, part_hbm,
               idxt_v, tblk_v, stage_v, lse_v, idxw_v, tgtw_v, pick_v,
               picked_v, acc_v, sem_a, sem_b, sem_m, sem_p):
    wid = lax.axis_index("s") * NC + lax.axis_index("c")

    pltpu.async_copy(idxt_hbm, idxt_v, sem_m)

    pltpu.sync_copy(idxf_hbm.at[wid], idxw_v)
    pltpu.sync_copy(tgtf_hbm.at[wid], tgtw_v)
    pltpu.sync_copy(lse_hbm, lse_v)

    def mkpick(j, _):
        i16 = idxw_v[pl.ds(j * L, L)]
        t16 = tgtw_v[pl.ds(j * L, L)]
        pick_v[pl.ds(j * L, L)] = t16 * V + i16
        return 0

    lax.fori_loop(0, P_PER_W // L, mkpick, 0)
    zero16 = jnp.zeros((L,), jnp.int32)
    for g in range((NPICK - P_PER_W) // L):
        pick_v[pl.ds(P_PER_W + g * L, L)] = zero16

    def pickgather(j, _):
        pltpu.async_copy(tflat_hbm.at[pick_v.at[pl.ds(j * 128, 128)]],
                         picked_v.at[pl.ds(j * 128, 128)], sem_p)
        return 0

    lax.fori_loop(0, NPICK // 128, pickgather, 0)

    def pickdrain(j, _):
        pltpu.make_async_copy(tflat_hbm.at[pick_v.at[pl.ds(j * 128, 128)]],
                              picked_v.at[pl.ds(j * 128, 128)], sem_p).wait()
        return 0

    lax.fori_loop(0, NPICK // 128, pickdrain, 0)

    def lossacc(j, acc):
        vids = idxw_v[pl.ds(j * L, L)]
        lz = plsc.load_gather(lse_v, [vids >> 7, vids & 127])
        pk = picked_v[pl.ds(j * L, L)]
        return acc + (lz - pk)

    acc = lax.fori_loop(0, P_PER_W // L, lossacc, jnp.zeros((L,), jnp.float32))
    acc_v[...] = acc
    pltpu.sync_copy(acc_v, part_hbm.at[wid])

    pltpu.make_async_copy(idxt_hbm, idxt_v, sem_m).wait()
    cr_lo = (wid * CR) // NW
    cr_hi = ((wid + 1) * CR) // NW

    def fill(t, buf):
        sb = stage_v.at[buf]
        irow = idxt_v.at[t]
        for bc in range(8):
            vids = [irow[pl.ds(bc * 128 + g * L, L)] for g in range(8)]
            prev = None
            for g in range(8):
                vals = []
                for ci in range(8):
                    vals.append(plsc.load_gather(tblk_v.at[ci], [vids[g]]))
                    if prev is not None:
                        sb[bc, ci, pl.ds((g - 1) * L, L)] = prev[ci]
                prev = vals
            for ci in range(8):
                sb[bc, ci, pl.ds(7 * L, L)] = prev[ci]

    def cr_body(cr, _):
        for ci in range(8):
            pltpu.sync_copy(tflat_hbm.at[pl.ds((cr * 8 + ci) * V, V)],
                            tblk_v.at[ci])
        kbase = (cr - cr_lo) * T

        def t_body(tp, _):
            for par, sem in ((0, sem_a), (1, sem_b)):
                t = 2 * tp + par
                k = kbase + t

                @pl.when(k >= 2)
                def _():
                    pltpu.make_async_copy(
                        stage_v.at[par], out_hbm.at[t, cr], sem).wait()

                fill(t, par)
                pltpu.async_copy(stage_v.at[par], out_hbm.at[t, cr], sem)
            return 0

        lax.fori_loop(0, T // 2, t_body, 0)
        return 0

    lax.fori_loop(cr_lo, cr_hi, cr_body, 0)
    last = cr_hi - 1
    pltpu.make_async_copy(stage_v.at[0], out_hbm.at[T - 2, last], sem_a).wait()
    pltpu.make_async_copy(stage_v.at[1], out_hbm.at[T - 1, last], sem_b).wait()


def kernel(idx, targets, token_embedding_table):
    idx = idx.astype(jnp.int32)
    targets = targets.astype(jnp.int32)
    lse = jnp.pad(_row_lse(token_embedding_table).reshape(V),
                  (0, 8 * 128 - V)).reshape(8, 128)
    ttab = token_embedding_table.T
    tflat = ttab.reshape(V * V)
    idxt = idx.T
    idxf = idx.reshape(NW, P_PER_W)
    tgtf = targets.reshape(NW, P_PER_W)
    out5, parts = _sc_writer(tflat, idxt, idxf, tgtf, lse)
    logits = jnp.transpose(out5, (2, 4, 0, 1, 3)).reshape(B, T, V)
    loss = jnp.sum(parts) / jnp.float32(BT)
    return (logits, loss)
